# Initial kernel scaffold; baseline (speedup 1.0000x reference)
import functools

import jax
import jax.numpy as jnp
from jax import lax
from jax.experimental import pallas as pl
from jax.experimental.pallas import tpu as pltpu
from jax.experimental.pallas import tpu_sc as plsc

BATCH = 16384
HIST = 200
EMBED_DIM = 32
N = BATCH * HIST

NUM_CORES = 2
NUM_SUBCORES = 16
NW = NUM_CORES * NUM_SUBCORES
PER_W = N // NW
CHUNK = 1024
NCHUNK = PER_W // CHUNK


def _make_gather():
    mesh = plsc.VectorSubcoreMesh(core_axis_name="c", subcore_axis_name="s")

    @functools.partial(
        pl.kernel,
        mesh=mesh,
        out_type=jax.ShapeDtypeStruct((N, EMBED_DIM), jnp.float32),
        scratch_types=[
            pltpu.VMEM((CHUNK,), jnp.int32),
            pltpu.VMEM((CHUNK, EMBED_DIM), jnp.float32),
            pltpu.SemaphoreType.DMA,
        ],
    )
    def gather_kernel(idx_hbm, table_hbm, out_hbm, idx_v, rows_v, sem):
        wid = lax.axis_index("s") * NUM_CORES + lax.axis_index("c")
        base = wid * PER_W

        def step(g, carry):
            off = base + g * CHUNK
            pltpu.sync_copy(idx_hbm.at[pl.ds(off, CHUNK)], idx_v)
            pltpu.async_copy(table_hbm.at[idx_v], rows_v, sem).wait()
            pltpu.sync_copy(rows_v, out_hbm.at[pl.ds(off, CHUNK)])
            return carry

        lax.fori_loop(0, NCHUNK, step, 0)

    return gather_kernel


_gather = _make_gather()


def kernel(x, embed_table):
    idx = x.reshape(N)
    out = _gather(idx, embed_table)
    return out.reshape(BATCH, HIST, EMBED_DIM)


# speedup vs baseline: 4.8095x; 4.8095x over previous
import functools

import jax
import jax.numpy as jnp
from jax import lax
from jax.experimental import pallas as pl
from jax.experimental.pallas import tpu as pltpu
from jax.experimental.pallas import tpu_sc as plsc

BATCH = 16384
HIST = 200
EMBED_DIM = 32
N = BATCH * HIST

NUM_CORES = 2
NUM_SUBCORES = 16
NW = NUM_CORES * NUM_SUBCORES
PER_W = N // NW
CHUNK = 1024
NCHUNK = PER_W // CHUNK


def _make_gather():
    mesh = plsc.VectorSubcoreMesh(core_axis_name="c", subcore_axis_name="s")

    @functools.partial(
        pl.kernel,
        mesh=mesh,
        out_type=jax.ShapeDtypeStruct((N, EMBED_DIM), jnp.float32),
        compiler_params=pltpu.CompilerParams(use_tc_tiling_on_sc=False),
        scratch_types=[
            pltpu.VMEM((CHUNK,), jnp.int32),
            pltpu.VMEM((CHUNK, EMBED_DIM), jnp.float32),
            pltpu.SemaphoreType.DMA,
        ],
    )
    def gather_kernel(idx_hbm, table_hbm, out_hbm, idx_v, rows_v, sem):
        wid = lax.axis_index("s") * NUM_CORES + lax.axis_index("c")
        base = wid * PER_W

        def step(g, carry):
            off = base + g * CHUNK
            pltpu.sync_copy(idx_hbm.at[pl.ds(off, CHUNK)], idx_v)
            pltpu.async_copy(table_hbm.at[idx_v], rows_v, sem).wait()
            pltpu.sync_copy(rows_v, out_hbm.at[pl.ds(off, CHUNK)])
            return carry

        lax.fori_loop(0, NCHUNK, step, 0)

    return gather_kernel


_gather = _make_gather()


def kernel(x, embed_table):
    idx = x.reshape(N)
    out = _gather(idx, embed_table)
    return out.reshape(BATCH, HIST, EMBED_DIM)


# speedup vs baseline: 5.0307x; 1.0460x over previous
import functools

import jax
import jax.numpy as jnp
from jax import lax
from jax.experimental import pallas as pl
from jax.experimental.pallas import tpu as pltpu
from jax.experimental.pallas import tpu_sc as plsc

BATCH = 16384
HIST = 200
EMBED_DIM = 32
N = BATCH * HIST

NUM_CORES = 2
NUM_SUBCORES = 16
NW = NUM_CORES * NUM_SUBCORES
PER_W = N // NW
CHUNK = 1600
NCHUNK = PER_W // CHUNK
NPAIR = NCHUNK // 2


def _make_gather():
    mesh = plsc.VectorSubcoreMesh(core_axis_name="c", subcore_axis_name="s")

    @functools.partial(
        pl.kernel,
        mesh=mesh,
        out_type=jax.ShapeDtypeStruct((N, EMBED_DIM), jnp.float32),
        compiler_params=pltpu.CompilerParams(use_tc_tiling_on_sc=False),
        scratch_types=[
            pltpu.VMEM((CHUNK,), jnp.int32),
            pltpu.VMEM((CHUNK,), jnp.int32),
            pltpu.VMEM((CHUNK, EMBED_DIM), jnp.float32),
            pltpu.VMEM((CHUNK, EMBED_DIM), jnp.float32),
            pltpu.SemaphoreType.DMA,
            pltpu.SemaphoreType.DMA,
            pltpu.SemaphoreType.DMA,
            pltpu.SemaphoreType.DMA,
        ],
    )
    def gather_kernel(
        idx_hbm, table_hbm, out_hbm,
        idx0, idx1, rows0, rows1,
        sem_g0, sem_g1, sem_o0, sem_o1,
    ):
        wid = lax.axis_index("s") * NUM_CORES + lax.axis_index("c")
        base = wid * PER_W

        def chunk_off(g):
            return base + g * CHUNK

        pltpu.sync_copy(idx_hbm.at[pl.ds(chunk_off(0), CHUNK)], idx0)
        pltpu.async_copy(table_hbm.at[idx0], rows0, sem_g0)

        def body(p, carry):
            g0 = 2 * p
            g1 = g0 + 1
            pltpu.sync_copy(idx_hbm.at[pl.ds(chunk_off(g1), CHUNK)], idx1)
            pltpu.make_async_copy(table_hbm.at[idx0], rows0, sem_g0).wait()
            pltpu.async_copy(table_hbm.at[idx1], rows1, sem_g1)
            pltpu.async_copy(
                rows0, out_hbm.at[pl.ds(chunk_off(g0), CHUNK)], sem_o0
            )

            @pl.when(p < NPAIR - 1)
            def _():
                pltpu.sync_copy(idx_hbm.at[pl.ds(chunk_off(g0 + 2), CHUNK)], idx0)

            pltpu.make_async_copy(
                rows0, out_hbm.at[pl.ds(chunk_off(g0), CHUNK)], sem_o0
            ).wait()

            @pl.when(p < NPAIR - 1)
            def _():
                pltpu.async_copy(table_hbm.at[idx0], rows0, sem_g0)

            pltpu.make_async_copy(table_hbm.at[idx1], rows1, sem_g1).wait()
            pltpu.async_copy(
                rows1, out_hbm.at[pl.ds(chunk_off(g1), CHUNK)], sem_o1
            )
            pltpu.make_async_copy(
                rows1, out_hbm.at[pl.ds(chunk_off(g1), CHUNK)], sem_o1
            ).wait()
            return carry

        lax.fori_loop(0, NPAIR, body, 0)

    return gather_kernel


_gather = _make_gather()


def kernel(x, embed_table):
    idx = x.reshape(N)
    out = _gather(idx, embed_table)
    return out.reshape(BATCH, HIST, EMBED_DIM)


# speedup vs baseline: 5.1547x; 1.0246x over previous
import functools

import jax
import jax.numpy as jnp
from jax import lax
from jax.experimental import pallas as pl
from jax.experimental.pallas import tpu as pltpu
from jax.experimental.pallas import tpu_sc as plsc

BATCH = 16384
HIST = 200
EMBED_DIM = 32
VOCAB = 1000000
N = BATCH * HIST
ROWS_PER_LINE = 4
LINES = VOCAB // ROWS_PER_LINE

NUM_CORES = 2
NUM_SUBCORES = 16
NW = NUM_CORES * NUM_SUBCORES
PER_W = N // NW
CHUNK = 160
NCHUNK = PER_W // CHUNK
NPAIR = NCHUNK // 2
GROUPS = CHUNK // 16


def _make_gather():
    mesh = plsc.VectorSubcoreMesh(core_axis_name="c", subcore_axis_name="s")

    @functools.partial(
        pl.kernel,
        mesh=mesh,
        out_type=jax.ShapeDtypeStruct((N, EMBED_DIM), jnp.float32),
        scratch_types=[
            pltpu.VMEM((CHUNK,), jnp.int32),
            pltpu.VMEM((CHUNK,), jnp.int32),
            pltpu.VMEM((CHUNK,), jnp.int32),
            pltpu.VMEM((CHUNK,), jnp.int32),
            pltpu.VMEM((CHUNK, 128), jnp.float32),
            pltpu.VMEM((CHUNK, 128), jnp.float32),
            pltpu.VMEM((CHUNK, EMBED_DIM), jnp.float32),
            pltpu.VMEM((CHUNK, EMBED_DIM), jnp.float32),
            pltpu.SemaphoreType.DMA,
            pltpu.SemaphoreType.DMA,
            pltpu.SemaphoreType.DMA,
            pltpu.SemaphoreType.DMA,
        ],
    )
    def gather_kernel(
        idx_hbm, table_lines, out_hbm,
        idxr0, idxr1, lidx0, lidx1, lines0, lines1, outv0, outv1,
        sem_g0, sem_g1, sem_o0, sem_o1,
    ):
        wid = lax.axis_index("s") * NUM_CORES + lax.axis_index("c")
        base = wid * PER_W
        iota = lax.iota(jnp.int32, 16)

        def chunk_off(g):
            return base + g * CHUNK

        def load_idx(g, idxr, lidx):
            pltpu.sync_copy(idx_hbm.at[pl.ds(chunk_off(g), CHUNK)], idxr)
            def lg(t, carry):
                j0 = t * 16
                v = idxr[pl.ds(j0, 16)]
                lidx[pl.ds(j0, 16)] = lax.shift_right_logical(v, 2)
                return carry
            lax.fori_loop(0, GROUPS, lg, 0)

        def extract(idxr, lines, outv):
            def grp(t, carry):
                j0 = t * 16
                qv = lax.shift_left(idxr[pl.ds(j0, 16)] & 3, 5)
                for li in range(16):
                    j = j0 + li
                    q32 = qv[li]
                    outv[j, pl.ds(0, 16)] = lines[j, pl.ds(q32, 16)]
                    outv[j, pl.ds(16, 16)] = lines[j, pl.ds(q32 + 16, 16)]
                return carry
            lax.fori_loop(0, GROUPS, grp, 0)

        load_idx(0, idxr0, lidx0)
        pltpu.async_copy(table_lines.at[lidx0], lines0, sem_g0)

        def body(p, carry):
            g0 = 2 * p
            g1 = g0 + 1
            load_idx(g1, idxr1, lidx1)
            pltpu.make_async_copy(table_lines.at[lidx0], lines0, sem_g0).wait()
            pltpu.async_copy(table_lines.at[lidx1], lines1, sem_g1)

            @pl.when(p > 0)
            def _():
                pltpu.make_async_copy(
                    outv0, out_hbm.at[pl.ds(chunk_off(g0), CHUNK)], sem_o0
                ).wait()

            extract(idxr0, lines0, outv0)
            pltpu.async_copy(
                outv0, out_hbm.at[pl.ds(chunk_off(g0), CHUNK)], sem_o0
            )

            @pl.when(p < NPAIR - 1)
            def _():
                load_idx(g0 + 2, idxr0, lidx0)

            pltpu.make_async_copy(table_lines.at[lidx1], lines1, sem_g1).wait()

            @pl.when(p < NPAIR - 1)
            def _():
                pltpu.async_copy(table_lines.at[lidx0], lines0, sem_g0)

            @pl.when(p > 0)
            def _():
                pltpu.make_async_copy(
                    outv1, out_hbm.at[pl.ds(chunk_off(g1), CHUNK)], sem_o1
                ).wait()

            extract(idxr1, lines1, outv1)
            pltpu.async_copy(
                outv1, out_hbm.at[pl.ds(chunk_off(g1), CHUNK)], sem_o1
            )
            return carry

        lax.fori_loop(0, NPAIR, body, 0)

        pltpu.make_async_copy(
            outv0, out_hbm.at[pl.ds(chunk_off(NCHUNK - 2), CHUNK)], sem_o0
        ).wait()
        pltpu.make_async_copy(
            outv1, out_hbm.at[pl.ds(chunk_off(NCHUNK - 1), CHUNK)], sem_o1
        ).wait()

    return gather_kernel


_gather = _make_gather()


def kernel(x, embed_table):
    idx = x.reshape(N)
    out = _gather(idx, embed_table.reshape(LINES, 128))
    return out.reshape(BATCH, HIST, EMBED_DIM)
